# baseline (speedup 1.0000x reference)
import functools

import jax
import jax.numpy as jnp
from jax import lax
from jax.experimental import pallas as pl
from jax.experimental.pallas import tpu as pltpu
from jax.experimental.pallas import tpu_sc as plsc

B, T = 64, 128
ROWS = B * T
IN_W = 543 * 3
OUT_W = 76 * 3
SHIFT = 1404

NUM_CORES = 2
NUM_SUBCORES = 16
NUM_WORKERS = NUM_CORES * NUM_SUBCORES
RPW = ROWS // NUM_WORKERS
CH = 64
NCHUNK = RPW // CH
GROUPS = CH // 16

LWX, LWY, RWX, RWY = 108 + SHIFT, 109 + SHIFT, 111 + SHIFT, 112 + SHIFT

_PLAN = []
for c in range(3, OUT_W):
    if c % 3 == 2:
        continue
    w = c - 3
    kind = 1 if w < 63 else (2 if w >= 162 else 0)
    _PLAN.append((c, kind, w + SHIFT))
_Z_COLS = list(range(2, OUT_W, 3))


def _sc_body(x_hbm, out_hbm, inbuf, outbuf):
    c = lax.axis_index("c")
    s = lax.axis_index("s")
    wid = s * NUM_CORES + c

    def chunk(k, carry):
        base = wid * RPW + k * CH

        pltpu.sync_copy(x_hbm.at[pl.ds(base, CH), :], inbuf)

        def group(g, carry2):
            rows = g * 16 + lax.iota(jnp.int32, 16)

            def cvec(v):
                return jnp.full((16,), v, jnp.int32)

            def gat(col):
                return plsc.load_gather(inbuf, [rows, cvec(col)])

            def scat(col, v):
                plsc.store_scatter(outbuf, [rows, cvec(col)], v)

            lwx, lwy = gat(LWX), gat(LWY)
            rwx, rwy = gat(RWX), gat(RWY)
            zero = jnp.zeros((16,), jnp.float32)
            scat(0, (lwx + rwx) * 0.5)
            scat(1, (lwy + rwy) * 0.5)
            for out_col, kind, src_col in _PLAN:
                d = gat(src_col)
                if kind == 1:
                    d = jnp.where(d != d, lwy if out_col % 3 == 1 else lwx, d)
                elif kind == 2:
                    d = jnp.where(d != d, rwy if out_col % 3 == 1 else rwx, d)
                scat(out_col, d)
            for col in _Z_COLS:
                scat(col, zero)
            return carry2

        lax.fori_loop(0, GROUPS, group, 0)

        pltpu.sync_copy(outbuf, out_hbm.at[pl.ds(base, CH), :])
        return carry

    lax.fori_loop(0, NCHUNK, chunk, 0)


_sc_call = functools.partial(
    pl.kernel,
    out_type=jax.ShapeDtypeStruct((ROWS, OUT_W), jnp.float32),
    mesh=plsc.VectorSubcoreMesh(core_axis_name="c", subcore_axis_name="s"),
    scratch_types=[pltpu.VMEM((CH, IN_W), jnp.float32),
                   pltpu.VMEM((CH, OUT_W), jnp.float32)],
    compiler_params=pltpu.CompilerParams(needs_layout_passes=False),
)(_sc_body)


@jax.jit
def kernel(keypoints):
    x = keypoints.reshape(ROWS, IN_W)
    out = _sc_call(x)
    return out.reshape(B, T, 76, 3)
